# baseline (speedup 1.0000x reference)
import functools

import jax
import jax.numpy as jnp
from jax import lax
from jax.experimental import pallas as pl
from jax.experimental.pallas import tpu as pltpu
from jax.experimental.pallas import tpu_sc as plsc

_L = 16
NW = 32
WVB = 4
WCAP = 64


def _make_phase_a(V, D, B):
    NB = 2 * B
    NVB = (V + 127) // 128
    vb_per_w = (NVB + NW - 1) // NW
    n_win = (vb_per_w + WVB - 1) // WVB
    DH = D // 8
    TRASH = NB

    mesh = plsc.VectorSubcoreMesh(core_axis_name="c", subcore_axis_name="s")

    @functools.partial(
        pl.kernel,
        out_type=jax.ShapeDtypeStruct((NB + _L, 2 * D), jnp.float32),
        mesh=mesh,
        scratch_types=[
            pltpu.VMEM((NB,), jnp.int32),
            pltpu.VMEM((NB + _L,), jnp.int32),
            pltpu.VMEM((DH, 8, WVB * 128), jnp.float32),
            pltpu.VMEM((WCAP, 2 * D), jnp.float32),
            pltpu.VMEM((1, WCAP), jnp.int32),
            pltpu.SMEM((1,), jnp.int32),
            pltpu.SemaphoreType.DMA,
            pltpu.SemaphoreType.DMA,
        ],
        compiler_params=pltpu.CompilerParams(needs_layout_passes=False),
    )
    def run(embt_hbm, idx_hbm, rows_hbm, idx_all, mj, sbuf, wrows, wj, msc, sem, sem2):
        wid = lax.axis_index("s") * 2 + lax.axis_index("c")
        vb0 = wid * vb_per_w
        vb_end = jnp.minimum(vb0 + vb_per_w, NVB)

        pltpu.sync_copy(idx_hbm, idx_all)
        iota = lax.iota(jnp.int32, _L)
        kq = [lax.shift_right_logical(c * _L + iota, 3) for c in range(D // _L)]
        lq = [lax.bitwise_and(c * _L + iota, 7) for c in range(D // _L)]

        def bin_body(t, n):
            vv = idx_all[pl.ds(t * _L, _L)]
            vb = lax.shift_right_logical(vv, 7)
            m = jnp.logical_and(vb >= vb0, vb < vb_end)
            pos = n + plsc.cumsum(m.astype(jnp.int32)) - 1
            plsc.store_scatter(mj, [pos], t * _L + iota, mask=m)
            return n + plsc.all_reduce_population_count(m)[0]

        n = lax.fori_loop(0, NB // _L, bin_body, jnp.int32(0), unroll=False)
        nvec = lax.shift_right_logical(n + _L - 1, 4)

        def win_body(win, carry):
            wb = vb0 + win * WVB
            valid = wb < vb_end

            @pl.when(win > 0)
            def _drain():
                pltpu.make_async_copy(
                    rows_hbm.at[pl.ds(0, WCAP)], wrows, sem2
                ).wait()

            msc[0] = 0

            @pl.when(valid)
            def _work():
                woff = pl.multiple_of(wb * 128, 128)
                copies = [
                    pltpu.make_async_copy(
                        embt_hbm.at[k, :, pl.ds(woff, WVB * 128)],
                        sbuf.at[k],
                        sem,
                    )
                    for k in range(DH)
                ]
                for cp in copies:
                    cp.start()
                for cp in copies:
                    cp.wait()

                def scan_body(u, carry2):
                    jv_raw = mj[pl.ds(u * _L, _L)]
                    jv = jnp.clip(jv_raw, 0, NB - 1)
                    vv = plsc.load_gather(idx_all, [jv])
                    lcol = vv - wb * 128
                    inb = jnp.logical_and(
                        jnp.logical_and(lcol >= 0, lcol < WVB * 128),
                        (u * _L + iota) < n,
                    )
                    pc = plsc.all_reduce_population_count(inb)[0]

                    @pl.when(pc > 0)
                    def _extract():
                        mc = msc[0]
                        pos = mc + plsc.cumsum(inb.astype(jnp.int32)) - 1
                        plsc.store_scatter(wj.at[0], [pos], jv, mask=inb)
                        inbi = inb.astype(jnp.int32)
                        for lane in range(_L):
                            @pl.when(inbi[lane] > 0)
                            def _one(lane=lane, lcol=lcol, pos=pos):
                                cvec = jnp.broadcast_to(lcol[lane], (_L,))
                                row = pos[lane]
                                for c in range(D // _L):
                                    val = plsc.load_gather(
                                        sbuf, [kq[c], lq[c], cvec]
                                    )
                                    wrows[row, pl.ds(c * _L, _L)] = val
                        msc[0] = mc + pc

                    return carry2

                lax.fori_loop(0, nvec, scan_body, 0, unroll=False)

            mc2 = msc[0]
            for b in range(WCAP // _L):
                ppos = mc2 + iota + b * _L
                pm = ppos < WCAP
                plsc.store_scatter(
                    wj.at[0], [ppos], jnp.full((_L,), TRASH, jnp.int32), mask=pm
                )
            pltpu.make_async_copy(wrows, rows_hbm.at[wj.at[0]], sem2).start()
            return carry

        lax.fori_loop(0, n_win, win_body, 0, unroll=False)
        pltpu.make_async_copy(rows_hbm.at[pl.ds(0, WCAP)], wrows, sem2).wait()

    return run


def _make_phase_b(V, D, B):
    bpw = B // NW
    n_rows = 2 * bpw
    HR = n_rows // 2
    n_grp = bpw // 2 // _L
    DC = D // _L

    mesh = plsc.VectorSubcoreMesh(core_axis_name="c", subcore_axis_name="s")

    @functools.partial(
        pl.kernel,
        out_type=jax.ShapeDtypeStruct((B,), jnp.float32),
        mesh=mesh,
        scratch_types=[
            pltpu.VMEM((HR, 2 * D), jnp.float32),
            pltpu.VMEM((D,), jnp.float32),
            pltpu.VMEM((_L, _L), jnp.float32),
            pltpu.VMEM((bpw,), jnp.float32),
            pltpu.SemaphoreType.DMA,
        ],
        compiler_params=pltpu.CompilerParams(needs_layout_passes=False),
    )
    def run(rows_hbm, r_hbm, out_hbm, rows_v, r_v, p_v, out_v, sem):
        wid = lax.axis_index("s") * 2 + lax.axis_index("c")
        pltpu.sync_copy(r_hbm, r_v)
        r_regs = [r_v[pl.ds(c * _L, _L)] for c in range(DC)]
        iota = lax.iota(jnp.int32, _L)

        for hs in range(2):
            base = wid * n_rows + hs * HR
            pltpu.async_copy(rows_hbm.at[pl.ds(base, HR)], rows_v, sem).wait()

            def group_body(g, carry):
                row0 = g * _L
                for j in range(_L):
                    i2 = 2 * (row0 + j)
                    acc = None
                    for c in range(DC):
                        s_c = rows_v[i2, pl.ds(c * _L, _L)]
                        o_c = rows_v[i2 + 1, pl.ds(c * _L, _L)]
                        t = (s_c * o_c) * r_regs[c]
                        acc = t if acc is None else acc + t
                    p_v[j, :] = acc
                accv = jnp.zeros((_L,), jnp.float32)
                for l in range(_L):
                    col = plsc.load_gather(p_v, [iota, jnp.full((_L,), l, jnp.int32)])
                    accv = accv + col
                sig = 1.0 / (1.0 + jnp.exp(-accv))
                out_v[pl.ds(hs * (bpw // 2) + g * _L, _L)] = sig
                return carry

            lax.fori_loop(0, n_grp, group_body, 0, unroll=False)

        pltpu.sync_copy(out_v, out_hbm.at[pl.ds(wid * bpw, bpw)])

    return run


def kernel(emb, batch_ind, r):
    V, D = emb.shape
    B = batch_ind.shape[0]
    embt3 = emb.T.reshape(D // 8, 8, V)
    idx_flat = batch_ind.reshape(2 * B)
    rows = _make_phase_a(V, D, B)(embt3, idx_flat)
    return _make_phase_b(V, D, B)(rows, r)


# speedup vs baseline: 1.0023x; 1.0023x over previous
import functools

import jax
import jax.numpy as jnp
from jax import lax
from jax.experimental import pallas as pl
from jax.experimental.pallas import tpu as pltpu
from jax.experimental.pallas import tpu_sc as plsc

_L = 16
NW = 32
WVB = 4
WCAP = 64


def _make_phase_a(V, D, B):
    NB = 2 * B
    NVB = (V + 127) // 128
    vb_per_w = (NVB + NW - 1) // NW
    n_win = (vb_per_w + WVB - 1) // WVB
    DH = D // 8
    TRASH = NB

    mesh = plsc.VectorSubcoreMesh(core_axis_name="c", subcore_axis_name="s")

    @functools.partial(
        pl.kernel,
        out_type=jax.ShapeDtypeStruct((NB + _L, 2 * D), jnp.float32),
        mesh=mesh,
        scratch_types=[
            pltpu.VMEM((NB,), jnp.int32),
            pltpu.VMEM((NB + _L,), jnp.int32),
            pltpu.VMEM((WVB, DH, 8, 128), jnp.float32),
            pltpu.VMEM((WCAP, 2 * D), jnp.float32),
            pltpu.VMEM((1, WCAP), jnp.int32),
            pltpu.SMEM((1,), jnp.int32),
            pltpu.SemaphoreType.DMA,
            pltpu.SemaphoreType.DMA,
        ],
        compiler_params=pltpu.CompilerParams(needs_layout_passes=False),
    )
    def run(embt_hbm, idx_hbm, rows_hbm, idx_all, mj, sbuf, wrows, wj, msc, sem, sem2):
        wid = lax.axis_index("s") * 2 + lax.axis_index("c")
        vb0 = wid * vb_per_w
        vb_end = jnp.minimum(vb0 + vb_per_w, NVB)

        pltpu.sync_copy(idx_hbm, idx_all)
        iota = lax.iota(jnp.int32, _L)
        kq = [lax.shift_right_logical(c * _L + iota, 3) for c in range(D // _L)]
        lq = [lax.bitwise_and(c * _L + iota, 7) for c in range(D // _L)]

        def bin_body(t, n):
            vv = idx_all[pl.ds(t * _L, _L)]
            vb = lax.shift_right_logical(vv, 7)
            m = jnp.logical_and(vb >= vb0, vb < vb_end)
            pos = n + plsc.cumsum(m.astype(jnp.int32)) - 1
            plsc.store_scatter(mj, [pos], t * _L + iota, mask=m)
            return n + plsc.all_reduce_population_count(m)[0]

        n = lax.fori_loop(0, NB // _L, bin_body, jnp.int32(0), unroll=False)
        nvec = lax.shift_right_logical(n + _L - 1, 4)

        def win_body(win, carry):
            wb = vb0 + win * WVB
            valid = wb < vb_end

            @pl.when(win > 0)
            def _drain():
                pltpu.make_async_copy(
                    rows_hbm.at[pl.ds(0, WCAP)], wrows, sem2
                ).wait()

            msc[0] = 0

            @pl.when(valid)
            def _work():
                copies = []
                for vbl in range(WVB):
                    woff = pl.multiple_of((wb + vbl) * 128, 128)
                    for k in range(DH):
                        copies.append(
                            pltpu.make_async_copy(
                                embt_hbm.at[k, :, pl.ds(woff, 128)],
                                sbuf.at[vbl, k],
                                sem,
                            )
                        )
                for cp in copies:
                    cp.start()
                for cp in copies:
                    cp.wait()

                def scan_body(u, carry2):
                    jv_raw = mj[pl.ds(u * _L, _L)]
                    jv = jnp.clip(jv_raw, 0, NB - 1)
                    vv = plsc.load_gather(idx_all, [jv])
                    lcol = vv - wb * 128
                    inb = jnp.logical_and(
                        jnp.logical_and(lcol >= 0, lcol < WVB * 128),
                        (u * _L + iota) < n,
                    )
                    pc = plsc.all_reduce_population_count(inb)[0]

                    @pl.when(pc > 0)
                    def _extract():
                        mc = msc[0]
                        pos = mc + plsc.cumsum(inb.astype(jnp.int32)) - 1
                        plsc.store_scatter(wj.at[0], [pos], jv, mask=inb)
                        inbi = inb.astype(jnp.int32)
                        for lane in range(_L):
                            @pl.when(inbi[lane] > 0)
                            def _one(lane=lane, lcol=lcol, pos=pos):
                                lc = lcol[lane]
                                vbvec = jnp.broadcast_to(
                                    lax.shift_right_logical(lc, 7), (_L,)
                                )
                                cvec = jnp.broadcast_to(
                                    lax.bitwise_and(lc, 127), (_L,)
                                )
                                row = pos[lane]
                                for c in range(D // _L):
                                    val = plsc.load_gather(
                                        sbuf, [vbvec, kq[c], lq[c], cvec]
                                    )
                                    wrows[row, pl.ds(c * _L, _L)] = val
                        msc[0] = mc + pc

                    return carry2

                lax.fori_loop(0, nvec, scan_body, 0, unroll=False)

            mc2 = msc[0]
            for b in range(WCAP // _L):
                ppos = mc2 + iota + b * _L
                pm = ppos < WCAP
                plsc.store_scatter(
                    wj.at[0], [ppos], jnp.full((_L,), TRASH, jnp.int32), mask=pm
                )
            pltpu.make_async_copy(wrows, rows_hbm.at[wj.at[0]], sem2).start()
            return carry

        lax.fori_loop(0, n_win, win_body, 0, unroll=False)
        pltpu.make_async_copy(rows_hbm.at[pl.ds(0, WCAP)], wrows, sem2).wait()

    return run


def _make_phase_b(V, D, B):
    bpw = B // NW
    n_rows = 2 * bpw
    HR = n_rows // 2
    n_grp = bpw // 2 // _L
    DC = D // _L

    mesh = plsc.VectorSubcoreMesh(core_axis_name="c", subcore_axis_name="s")

    @functools.partial(
        pl.kernel,
        out_type=jax.ShapeDtypeStruct((B,), jnp.float32),
        mesh=mesh,
        scratch_types=[
            pltpu.VMEM((HR, 2 * D), jnp.float32),
            pltpu.VMEM((D,), jnp.float32),
            pltpu.VMEM((_L, _L), jnp.float32),
            pltpu.VMEM((bpw,), jnp.float32),
            pltpu.SemaphoreType.DMA,
        ],
        compiler_params=pltpu.CompilerParams(needs_layout_passes=False),
    )
    def run(rows_hbm, r_hbm, out_hbm, rows_v, r_v, p_v, out_v, sem):
        wid = lax.axis_index("s") * 2 + lax.axis_index("c")
        pltpu.sync_copy(r_hbm, r_v)
        r_regs = [r_v[pl.ds(c * _L, _L)] for c in range(DC)]
        iota = lax.iota(jnp.int32, _L)

        for hs in range(2):
            base = wid * n_rows + hs * HR
            pltpu.async_copy(rows_hbm.at[pl.ds(base, HR)], rows_v, sem).wait()

            def group_body(g, carry):
                row0 = g * _L
                for j in range(_L):
                    i2 = 2 * (row0 + j)
                    acc = None
                    for c in range(DC):
                        s_c = rows_v[i2, pl.ds(c * _L, _L)]
                        o_c = rows_v[i2 + 1, pl.ds(c * _L, _L)]
                        t = (s_c * o_c) * r_regs[c]
                        acc = t if acc is None else acc + t
                    p_v[j, :] = acc
                accv = jnp.zeros((_L,), jnp.float32)
                for l in range(_L):
                    col = plsc.load_gather(p_v, [iota, jnp.full((_L,), l, jnp.int32)])
                    accv = accv + col
                sig = 1.0 / (1.0 + jnp.exp(-accv))
                out_v[pl.ds(hs * (bpw // 2) + g * _L, _L)] = sig
                return carry

            lax.fori_loop(0, n_grp, group_body, 0, unroll=False)

        pltpu.sync_copy(out_v, out_hbm.at[pl.ds(wid * bpw, bpw)])

    return run


def kernel(emb, batch_ind, r):
    V, D = emb.shape
    B = batch_ind.shape[0]
    embt3 = emb.T.reshape(D // 8, 8, V)
    idx_flat = batch_ind.reshape(2 * B)
    rows = _make_phase_a(V, D, B)(embt3, idx_flat)
    return _make_phase_b(V, D, B)(rows, r)
